# baseline (speedup 1.0000x reference)
import functools

import jax
import jax.numpy as jnp
from jax import lax
from jax.experimental import pallas as pl
from jax.experimental.pallas import tpu as pltpu
from jax.experimental.pallas import tpu_sc as plsc

BATCH = 16384
EMBED_DIM = 64

_INFO = plsc.get_sparse_core_info()
_NC = _INFO.num_cores
_NS = _INFO.num_subcores
_L = _INFO.num_lanes
_NW = _NC * _NS
_BPW = BATCH // _NW
_CHUNK = 128
_NCHUNK = _BPW // _CHUNK


@functools.partial(
    pl.kernel,
    mesh=plsc.VectorSubcoreMesh(core_axis_name="c", subcore_axis_name="s"),
    compiler_params=pltpu.CompilerParams(
        needs_layout_passes=False, use_tc_tiling_on_sc=False),
    out_type=jax.ShapeDtypeStruct((BATCH,), jnp.float32),
    scratch_types=[
        pltpu.VMEM((_NCHUNK, _CHUNK), jnp.int32),
        pltpu.VMEM((_NCHUNK, _CHUNK), jnp.int32),
        pltpu.VMEM((_BPW, EMBED_DIM), jnp.float32),
        pltpu.VMEM((_BPW, EMBED_DIM), jnp.float32),
        pltpu.VMEM((_BPW,), jnp.float32),
        pltpu.SemaphoreType.DMA,
    ],
)
def _sc_dot_kernel(uids_hbm, mids_hbm, utab_hbm, mtab_hbm, out_hbm,
                   uidx_v, midx_v, urows_v, mrows_v, out_v, sem):
    wid = lax.axis_index("s") * _NC + lax.axis_index("c")
    base = wid * _BPW

    pltpu.sync_copy(uids_hbm.at[pl.ds(wid * _NCHUNK, _NCHUNK)], uidx_v)
    pltpu.sync_copy(mids_hbm.at[pl.ds(wid * _NCHUNK, _NCHUNK)], midx_v)

    copies = []
    for j in range(_NCHUNK):
        dst_u = urows_v.at[pl.ds(j * _CHUNK, _CHUNK)]
        dst_m = mrows_v.at[pl.ds(j * _CHUNK, _CHUNK)]
        copies.append(pltpu.async_copy(utab_hbm.at[uidx_v.at[j]], dst_u, sem))
        copies.append(pltpu.async_copy(mtab_hbm.at[midx_v.at[j]], dst_m, sem))
    for c in copies:
        c.wait()

    def group_body(g, carry):
        rows = g * _L + lax.iota(jnp.int32, _L)
        acc = jnp.zeros((_L,), jnp.float32)
        for j in range(EMBED_DIM):
            cols = jnp.full((_L,), j, jnp.int32)
            u = plsc.load_gather(urows_v, [rows, cols])
            m = plsc.load_gather(mrows_v, [rows, cols])
            acc = acc + u * m
        out_v[pl.ds(g * _L, _L)] = acc
        return carry

    lax.fori_loop(0, _BPW // _L, group_body, 0)

    pltpu.sync_copy(out_v, out_hbm.at[pl.ds(base, _BPW)])


def kernel(user_ids, movie_ids, user_table, movie_table):
    uids = user_ids.astype(jnp.int32).reshape(_NW * _NCHUNK, _CHUNK)
    mids = movie_ids.astype(jnp.int32).reshape(_NW * _NCHUNK, _CHUNK)
    return _sc_dot_kernel(uids, mids, user_table, movie_table)
